# Initial kernel scaffold; baseline (speedup 1.0000x reference)
import jax
import jax.numpy as jnp
from jax.experimental import pallas as pl


def kernel(x_user, x_item, params, edge_index_u2i, edge_index_i2u):
    raise NotImplementedError("write your pallas kernel here")


# speedup vs baseline: 16.1560x; 16.1560x over previous
import functools

import jax
import jax.numpy as jnp
from jax import lax
from jax.experimental import pallas as pl
from jax.experimental.pallas import tpu as pltpu
from jax.experimental.pallas import tpu_sc as plsc

N = 50000
D = 64
NTILE = 16
IB = 125
IR = 8
CHUNK = IB * IR

def _mesh():
    return plsc.VectorSubcoreMesh(core_axis_name="c", subcore_axis_name="s")


_SC_PARAMS = pltpu.CompilerParams(use_tc_tiling_on_sc=False)


def _sc_degrees(srcs, dsts, ones, zeros, E):
    rows_per_tile = E // (NTILE * IB)
    n_chunks = rows_per_tile // IR

    @functools.partial(
        pl.kernel,
        mesh=_mesh(),
        compiler_params=_SC_PARAMS,
        out_type=jax.ShapeDtypeStruct((4 * N,), jnp.float32),
        scratch_types=[
            pltpu.VMEM((IR, IB), jnp.int32),
            pltpu.VMEM((IR, IB), jnp.int32),
            pltpu.VMEM((IR, IB), jnp.float32),
            pltpu.VMEM((2000,), jnp.float32),
            pltpu.VMEM_SHARED((2 * N,), jnp.float32),
        ],
    )
    def k(srcs_hbm, dsts_hbm, ones_hbm, zeros_hbm, out_hbm, sidx, didx, ones_v,
          zv, acc):
        c = lax.axis_index("c")
        s = lax.axis_index("s")
        pltpu.sync_copy(ones_hbm, ones_v)
        pltpu.sync_copy(zeros_hbm, zv)

        @pl.when(s < 4)
        def _():
            base = s * (2 * N // 4)
            for kk in range(12):
                pltpu.sync_copy(zv, acc.at[pl.ds(base + kk * 2000, 2000)])
            pltpu.sync_copy(zv.at[pl.ds(0, 1000)],
                            acc.at[pl.ds(base + 24000, 1000)])

        plsc.subcore_barrier()

        def chunk(i, carry):
            r0 = s * rows_per_tile + i * IR
            pltpu.sync_copy(srcs_hbm.at[c, pl.ds(r0, IR)], sidx)
            pltpu.sync_copy(dsts_hbm.at[c, pl.ds(r0, IR)], didx)
            for j in range(IR):
                pltpu.sync_copy(ones_v.at[j], acc.at[sidx.at[j]], add=True)
                pltpu.sync_copy(ones_v.at[j], acc.at[didx.at[j]], add=True)
            return carry

        lax.fori_loop(0, n_chunks, chunk, 0)
        plsc.subcore_barrier()

        @pl.when(s < 4)
        def _():
            base = s * (2 * N // 4)
            for kk in range(12):
                pltpu.sync_copy(acc.at[pl.ds(base + kk * 2000, 2000)], zv)
                pltpu.sync_copy(
                    zv, out_hbm.at[pl.ds(c * 2 * N + base + kk * 2000, 2000)])
            pltpu.sync_copy(acc.at[pl.ds(base + 24000, 1000)],
                            zv.at[pl.ds(0, 1000)])
            pltpu.sync_copy(zv.at[pl.ds(0, 1000)],
                            out_hbm.at[pl.ds(c * 2 * N + base + 24000, 1000)])

    return k(srcs, dsts, ones, zeros)


NP = 50048


def _sc_scatter(table, srcs, dsts, zeros, W, E, dst_per_core):
    rows_per_tile = E // (NTILE * IB)
    n_chunks = rows_per_tile // IR
    rpt = NP // NTILE

    @functools.partial(
        pl.kernel,
        mesh=_mesh(),
        compiler_params=_SC_PARAMS,
        out_type=jax.ShapeDtypeStruct((2, NP, W), jnp.float32),
        scratch_types=[
            pltpu.VMEM((IR, IB), jnp.int32),
            pltpu.VMEM((IR, IB), jnp.int32),
            pltpu.VMEM((IR, IB, W), jnp.float32),
            pltpu.VMEM((1000, W), jnp.float32),
            pltpu.VMEM_SHARED((NP, W), jnp.float32),
            pltpu.SemaphoreType.DMA,
        ],
    )
    def k(table_hbm, srcs_hbm, dsts_hbm, zeros_hbm, out_hbm, sidx, didx, rows,
          zbuf, acc, sem):
        c = lax.axis_index("c")
        s = lax.axis_index("s")
        base = s * rpt
        pltpu.sync_copy(zeros_hbm, zbuf)
        for kk in range(3):
            pltpu.sync_copy(zbuf, acc.at[pl.ds(base + kk * 1000, 1000)])
        pltpu.sync_copy(zbuf.at[pl.ds(0, 128)], acc.at[pl.ds(base + 3000, 128)])
        plsc.subcore_barrier()

        def chunk(i, carry):
            r0 = s * rows_per_tile + i * IR
            pltpu.sync_copy(srcs_hbm.at[c, pl.ds(r0, IR)], sidx)
            if dst_per_core:
                pltpu.sync_copy(dsts_hbm.at[c, pl.ds(r0, IR)], didx)
            else:
                pltpu.sync_copy(dsts_hbm.at[pl.ds(r0, IR)], didx)
            cps = [
                pltpu.async_copy(table_hbm.at[sidx.at[j]], rows.at[j], sem)
                for j in range(IR)
            ]
            for cp in cps:
                cp.wait()
            for j in range(IR):
                pltpu.sync_copy(rows.at[j], acc.at[didx.at[j]], add=True)
            return carry

        lax.fori_loop(0, n_chunks, chunk, 0)
        plsc.subcore_barrier()
        for kk in range(3):
            pltpu.sync_copy(acc.at[pl.ds(base + kk * 1000, 1000)], zbuf)
            pltpu.sync_copy(zbuf, out_hbm.at[c, pl.ds(base + kk * 1000, 1000)])
        pltpu.sync_copy(acc.at[pl.ds(base + 3000, 128)], zbuf.at[pl.ds(0, 128)])
        pltpu.sync_copy(zbuf.at[pl.ds(0, 128)],
                        out_hbm.at[c, pl.ds(base + 3000, 128)])

    return k(table, srcs, dsts, zeros)


_BLK = 2000


def _ln_in(x, g, b):
    mu = jnp.mean(x, axis=-1, keepdims=True)
    var = jnp.mean((x - mu) ** 2, axis=-1, keepdims=True)
    return (x - mu) * lax.rsqrt(var + 1e-5) * g + b


def _enc_body(x_ref, win_ref, w1s_ref, w2s_ref, vecs_ref, wout_ref, wg_ref,
              deg_ref, out_ref):
    f32 = jnp.float32
    x = x_ref[...]
    v = vecs_ref
    h = jnp.dot(x, win_ref[...], preferred_element_type=f32) + v[0]
    for i in range(4):
        z = _ln_in(h, v[1 + i], v[5 + i])
        z = jnp.maximum(jnp.dot(z, w1s_ref[i], preferred_element_type=f32) + v[9 + i], 0.0)
        h = h + jnp.dot(z, w2s_ref[i], preferred_element_type=f32) + v[13 + i]
    h = jnp.maximum(_ln_in(h, v[17], v[18]), 0.0)
    h = jnp.dot(h, wout_ref[...], preferred_element_type=f32) + v[19]
    dsc = lax.rsqrt(jnp.maximum(deg_ref[...], 1.0))
    hw = jnp.dot(h * dsc, wg_ref[...], preferred_element_type=f32)
    for q in range(4):
        out_ref[q] = hw[:, 16 * q:16 * (q + 1)]


def _encode_project(x, enc, deg, wg):
    vecs = jnp.stack(
        [enc["b_in"]]
        + [b["g"] for b in enc["blocks"]]
        + [b["bt"] for b in enc["blocks"]]
        + [b["b1"] for b in enc["blocks"]]
        + [b["b2"] for b in enc["blocks"]]
        + [enc["g_out"], enc["bt_out"], enc["b_out"]]
    )
    vecs = jnp.pad(vecs, ((0, 4), (0, 0)))
    w1s = jnp.stack([b["W1"] for b in enc["blocks"]])
    w2s = jnp.stack([b["W2"] for b in enc["blocks"]])
    g = N // _BLK
    full = lambda a: pl.BlockSpec(a.shape, lambda i: (0,) * a.ndim)
    return pl.pallas_call(
        _enc_body,
        grid=(g,),
        in_specs=[
            pl.BlockSpec((_BLK, x.shape[1]), lambda i: (i, 0)),
            full(enc["W_in"]), full(w1s), full(w2s), full(vecs),
            full(enc["W_out"]), full(wg),
            pl.BlockSpec((_BLK, 1), lambda i: (i, 0)),
        ],
        out_specs=pl.BlockSpec((4, _BLK, 16), lambda i: (0, i, 0)),
        out_shape=jax.ShapeDtypeStruct((4, N, 16), jnp.float32),
    )(x, enc["W_in"], w1s, w2s, vecs, enc["W_out"], wg, deg)


def _l2_body(acc_ref, dega_ref, degb_ref, b1_ref, w2_ref, out_ref):
    dsa = lax.rsqrt(jnp.maximum(dega_ref[...], 1.0))
    dsb = lax.rsqrt(jnp.maximum(degb_ref[...], 1.0))
    b1 = b1_ref[0:1]
    w2 = w2_ref[...]
    y = jnp.zeros(out_ref.shape, jnp.float32)
    for q in range(4):
        uq = jnp.maximum(acc_ref[q] * dsa + b1[:, 16 * q:16 * (q + 1)], 0.0)
        y = y + jnp.dot(uq, w2[16 * q:16 * (q + 1)],
                        preferred_element_type=jnp.float32)
    out_ref[...] = y * dsb


def _l2_prep(acc, deg_relu, deg_scat, b1, w2pad):
    b1p = jnp.pad(b1[None, :], ((0, 7), (0, 0)))
    g = N // _BLK
    full = lambda a: pl.BlockSpec(a.shape, lambda i: (0,) * a.ndim)
    return pl.pallas_call(
        _l2_body,
        grid=(g,),
        in_specs=[
            pl.BlockSpec((4, _BLK, 16), lambda i: (0, i, 0)),
            pl.BlockSpec((_BLK, 1), lambda i: (i, 0)),
            pl.BlockSpec((_BLK, 1), lambda i: (i, 0)),
            full(b1p), full(w2pad),
        ],
        out_specs=pl.BlockSpec((_BLK, 16), lambda i: (i, 0)),
        out_shape=jax.ShapeDtypeStruct((N, 16), jnp.float32),
    )(acc, deg_relu, deg_scat, b1p, w2pad)


def _fin_body(acc2_ref, degu_ref, degi_ref, b2_ref, u_ref, i_ref):
    dsu = lax.rsqrt(jnp.maximum(degu_ref[...], 1.0))
    dsi = lax.rsqrt(jnp.maximum(degi_ref[...], 1.0))
    b2 = b2_ref[...]
    u_ref[...] = acc2_ref[0][:, :2] * dsu + b2[0:1, :2]
    i_ref[...] = acc2_ref[1][:, :2] * dsi + b2[1:2, :2]


def _finalize(acc2, deg_u, deg_i, b2u, b2i):
    b2 = jnp.pad(jnp.stack([b2u, b2i]), ((0, 6), (0, 0)))
    g = N // _BLK
    full = lambda a: pl.BlockSpec(a.shape, lambda i: (0,) * a.ndim)
    return pl.pallas_call(
        _fin_body,
        grid=(g,),
        in_specs=[
            pl.BlockSpec((2, _BLK, 16), lambda i: (0, i, 0)),
            pl.BlockSpec((_BLK, 1), lambda i: (i, 0)),
            pl.BlockSpec((_BLK, 1), lambda i: (i, 0)),
            full(b2),
        ],
        out_specs=[
            pl.BlockSpec((_BLK, 2), lambda i: (i, 0)),
            pl.BlockSpec((_BLK, 2), lambda i: (i, 0)),
        ],
        out_shape=[
            jax.ShapeDtypeStruct((N, 2), jnp.float32),
            jax.ShapeDtypeStruct((N, 2), jnp.float32),
        ],
    )(acc2, deg_u, deg_i, b2)


def kernel(x_user, x_item, params, edge_index_u2i, edge_index_i2u):
    p = params
    E = edge_index_u2i.shape[1]
    i32 = jnp.int32
    su_u = edge_index_u2i[0].astype(i32)
    di_u = edge_index_u2i[1].astype(i32)
    si_i = edge_index_i2u[0].astype(i32)
    du_i = edge_index_i2u[1].astype(i32)

    rows3 = lambda a: a.reshape(2, E // IB, IB)
    rows2 = lambda a: a.reshape(E // IB, IB)

    ones = jnp.ones((IR, IB), jnp.float32)
    zeros_deg = jnp.zeros((2000,), jnp.float32)
    degs = _sc_degrees(
        rows3(jnp.stack([su_u, si_i])),
        rows3(jnp.stack([di_u, du_i]) + N),
        ones, zeros_deg, E,
    ).reshape(2, 2 * N)
    deg_src_u2i = degs[0, :N].reshape(N, 1)
    deg_dst_u2i = degs[0, N:].reshape(N, 1)
    deg_src_i2u = degs[1, :N].reshape(N, 1)
    deg_dst_i2u = degs[1, N:].reshape(N, 1)

    t_i2u = _encode_project(x_item, p["enc_item"], deg_src_i2u, p["l1_i2u_W"])
    t_u2i = _encode_project(x_user, p["enc_user"], deg_src_u2i, p["l1_u2i_W"])

    zeros16 = jnp.zeros((1000, 16), jnp.float32)

    def scat64(table4, src, dst):
        tab = table4.reshape(4 * N, 16)
        dst_r = rows2(dst)
        parts = [
            _sc_scatter(tab, rows3(jnp.stack([src + 2 * q * N,
                                              src + (2 * q + 1) * N])),
                        dst_r, zeros16, 16, E, dst_per_core=False)[:, :N]
            for q in range(2)
        ]
        return jnp.concatenate(parts, axis=0)

    acc_u = scat64(t_i2u, si_i, du_i)
    acc_i = scat64(t_u2i, su_u, di_u)

    pad16 = lambda w: jnp.pad(w, ((0, 0), (0, 14)))
    y_i = _l2_prep(acc_i, deg_dst_u2i, deg_src_i2u, p["l1_u2i_b"],
                   pad16(p["l2_i2u_W"]))
    y_u = _l2_prep(acc_u, deg_dst_i2u, deg_src_u2i, p["l1_i2u_b"],
                   pad16(p["l2_u2i_W"]))

    acc2 = _sc_scatter(jnp.concatenate([y_i, y_u], axis=0),
                       rows3(jnp.stack([si_i, su_u + N])),
                       rows3(jnp.stack([du_i, di_u])),
                       zeros16, 16, E, dst_per_core=True)[:, :N]

    u2, i2 = _finalize(acc2, deg_dst_i2u, deg_dst_u2i,
                       p["l2_i2u_b"], p["l2_u2i_b"])
    return (u2, i2)


# speedup vs baseline: 19.6325x; 1.2152x over previous
import functools

import jax
import jax.numpy as jnp
from jax import lax
from jax.experimental import pallas as pl
from jax.experimental.pallas import tpu as pltpu
from jax.experimental.pallas import tpu_sc as plsc

N = 50000
D = 64
NTILE = 16
IB = 125
IR = 8
CHUNK = IB * IR

def _mesh():
    return plsc.VectorSubcoreMesh(core_axis_name="c", subcore_axis_name="s")


_SC_PARAMS = pltpu.CompilerParams(use_tc_tiling_on_sc=False)


def _sc_degrees(srcs, dsts, ones, zeros, E):
    rows_per_tile = E // (NTILE * IB)
    n_chunks = rows_per_tile // IR

    @functools.partial(
        pl.kernel,
        mesh=_mesh(),
        compiler_params=_SC_PARAMS,
        out_type=jax.ShapeDtypeStruct((4 * N,), jnp.float32),
        scratch_types=[
            pltpu.VMEM((IR, IB), jnp.int32),
            pltpu.VMEM((IR, IB), jnp.int32),
            pltpu.VMEM((IR, IB), jnp.float32),
            pltpu.VMEM((2000,), jnp.float32),
            pltpu.VMEM_SHARED((2 * N,), jnp.float32),
        ],
    )
    def k(srcs_hbm, dsts_hbm, ones_hbm, zeros_hbm, out_hbm, sidx, didx, ones_v,
          zv, acc):
        c = lax.axis_index("c")
        s = lax.axis_index("s")
        pltpu.sync_copy(ones_hbm, ones_v)
        pltpu.sync_copy(zeros_hbm, zv)

        @pl.when(s < 4)
        def _():
            base = s * (2 * N // 4)
            for kk in range(12):
                pltpu.sync_copy(zv, acc.at[pl.ds(base + kk * 2000, 2000)])
            pltpu.sync_copy(zv.at[pl.ds(0, 1000)],
                            acc.at[pl.ds(base + 24000, 1000)])

        plsc.subcore_barrier()

        def chunk(i, carry):
            r0 = s * rows_per_tile + i * IR
            pltpu.sync_copy(srcs_hbm.at[c, pl.ds(r0, IR)], sidx)
            pltpu.sync_copy(dsts_hbm.at[c, pl.ds(r0, IR)], didx)
            for j in range(IR):
                pltpu.sync_copy(ones_v.at[j], acc.at[sidx.at[j]], add=True)
                pltpu.sync_copy(ones_v.at[j], acc.at[didx.at[j]], add=True)
            return carry

        lax.fori_loop(0, n_chunks, chunk, 0)
        plsc.subcore_barrier()

        @pl.when(s < 4)
        def _():
            base = s * (2 * N // 4)
            for kk in range(12):
                pltpu.sync_copy(acc.at[pl.ds(base + kk * 2000, 2000)], zv)
                pltpu.sync_copy(
                    zv, out_hbm.at[pl.ds(c * 2 * N + base + kk * 2000, 2000)])
            pltpu.sync_copy(acc.at[pl.ds(base + 24000, 1000)],
                            zv.at[pl.ds(0, 1000)])
            pltpu.sync_copy(zv.at[pl.ds(0, 1000)],
                            out_hbm.at[pl.ds(c * 2 * N + base + 24000, 1000)])

    return k(srcs, dsts, ones, zeros)


NP = 50048


def _sc_scatter(table, srcs, dsts, zeros, W, E, dst_per_core):
    rows_per_tile = E // (NTILE * IB)
    n_chunks = rows_per_tile // IR
    rpt = NP // NTILE

    @functools.partial(
        pl.kernel,
        mesh=_mesh(),
        compiler_params=_SC_PARAMS,
        out_type=jax.ShapeDtypeStruct((2, NP, W), jnp.float32),
        scratch_types=[
            pltpu.VMEM((2, IR, IB), jnp.int32),
            pltpu.VMEM((2, IR, IB), jnp.int32),
            pltpu.VMEM((2, IR, IB, W), jnp.float32),
            pltpu.VMEM((1000, W), jnp.float32),
            pltpu.VMEM_SHARED((NP, W), jnp.float32),
            pltpu.SemaphoreType.DMA,
            pltpu.SemaphoreType.DMA,
        ],
    )
    def k(table_hbm, srcs_hbm, dsts_hbm, zeros_hbm, out_hbm, sidx, didx, rows,
          zbuf, acc, semA, semB):
        c = lax.axis_index("c")
        s = lax.axis_index("s")
        base = s * rpt
        pltpu.sync_copy(zeros_hbm, zbuf)
        for kk in range(3):
            pltpu.sync_copy(zbuf, acc.at[pl.ds(base + kk * 1000, 1000)])
        pltpu.sync_copy(zbuf.at[pl.ds(0, 128)], acc.at[pl.ds(base + 3000, 128)])
        plsc.subcore_barrier()

        sems = (semA, semB)
        e0 = s * rows_per_tile

        def load_idx(i, b):
            r0 = e0 + i * IR
            pltpu.sync_copy(srcs_hbm.at[c, pl.ds(r0, IR)], sidx.at[b])
            if dst_per_core:
                pltpu.sync_copy(dsts_hbm.at[c, pl.ds(r0, IR)], didx.at[b])
            else:
                pltpu.sync_copy(dsts_hbm.at[pl.ds(r0, IR)], didx.at[b])

        def fire(b):
            for j in range(IR):
                pltpu.async_copy(table_hbm.at[sidx.at[b, j]], rows.at[b, j],
                                 sems[b])

        def drain_scatter(b):
            for j in range(IR):
                pltpu.make_async_copy(table_hbm.at[sidx.at[b, j]],
                                      rows.at[b, j], sems[b]).wait()
            for j in range(IR):
                pltpu.sync_copy(rows.at[b, j], acc.at[didx.at[b, j]], add=True)

        load_idx(0, 0)
        fire(0)

        def step(i, carry):
            load_idx(2 * i + 1, 1)
            fire(1)
            drain_scatter(0)

            @pl.when(i < n_chunks // 2 - 1)
            def _():
                load_idx(2 * i + 2, 0)
                fire(0)

            drain_scatter(1)
            return carry

        lax.fori_loop(0, n_chunks // 2, step, 0)
        plsc.subcore_barrier()
        for kk in range(3):
            pltpu.sync_copy(acc.at[pl.ds(base + kk * 1000, 1000)], zbuf)
            pltpu.sync_copy(zbuf, out_hbm.at[c, pl.ds(base + kk * 1000, 1000)])
        pltpu.sync_copy(acc.at[pl.ds(base + 3000, 128)], zbuf.at[pl.ds(0, 128)])
        pltpu.sync_copy(zbuf.at[pl.ds(0, 128)],
                        out_hbm.at[c, pl.ds(base + 3000, 128)])

    return k(table, srcs, dsts, zeros)


_BLK = 2000


def _ln_in(x, g, b):
    mu = jnp.mean(x, axis=-1, keepdims=True)
    var = jnp.mean((x - mu) ** 2, axis=-1, keepdims=True)
    return (x - mu) * lax.rsqrt(var + 1e-5) * g + b


def _enc_body(x_ref, win_ref, w1s_ref, w2s_ref, vecs_ref, wout_ref, wg_ref,
              deg_ref, out_ref):
    f32 = jnp.float32
    x = x_ref[...]
    v = vecs_ref
    h = jnp.dot(x, win_ref[...], preferred_element_type=f32) + v[0]
    for i in range(4):
        z = _ln_in(h, v[1 + i], v[5 + i])
        z = jnp.maximum(jnp.dot(z, w1s_ref[i], preferred_element_type=f32) + v[9 + i], 0.0)
        h = h + jnp.dot(z, w2s_ref[i], preferred_element_type=f32) + v[13 + i]
    h = jnp.maximum(_ln_in(h, v[17], v[18]), 0.0)
    h = jnp.dot(h, wout_ref[...], preferred_element_type=f32) + v[19]
    dsc = lax.rsqrt(jnp.maximum(deg_ref[...], 1.0))
    hw = jnp.dot(h * dsc, wg_ref[...], preferred_element_type=f32)
    for q in range(4):
        out_ref[q] = hw[:, 16 * q:16 * (q + 1)]


def _encode_project(x, enc, deg, wg):
    vecs = jnp.stack(
        [enc["b_in"]]
        + [b["g"] for b in enc["blocks"]]
        + [b["bt"] for b in enc["blocks"]]
        + [b["b1"] for b in enc["blocks"]]
        + [b["b2"] for b in enc["blocks"]]
        + [enc["g_out"], enc["bt_out"], enc["b_out"]]
    )
    vecs = jnp.pad(vecs, ((0, 4), (0, 0)))
    w1s = jnp.stack([b["W1"] for b in enc["blocks"]])
    w2s = jnp.stack([b["W2"] for b in enc["blocks"]])
    g = N // _BLK
    full = lambda a: pl.BlockSpec(a.shape, lambda i: (0,) * a.ndim)
    return pl.pallas_call(
        _enc_body,
        grid=(g,),
        in_specs=[
            pl.BlockSpec((_BLK, x.shape[1]), lambda i: (i, 0)),
            full(enc["W_in"]), full(w1s), full(w2s), full(vecs),
            full(enc["W_out"]), full(wg),
            pl.BlockSpec((_BLK, 1), lambda i: (i, 0)),
        ],
        out_specs=pl.BlockSpec((4, _BLK, 16), lambda i: (0, i, 0)),
        out_shape=jax.ShapeDtypeStruct((4, N, 16), jnp.float32),
    )(x, enc["W_in"], w1s, w2s, vecs, enc["W_out"], wg, deg)


def _l2_body(acc_ref, dega_ref, degb_ref, b1_ref, w2_ref, out_ref):
    dsa = lax.rsqrt(jnp.maximum(dega_ref[...], 1.0))
    dsb = lax.rsqrt(jnp.maximum(degb_ref[...], 1.0))
    b1 = b1_ref[0:1]
    w2 = w2_ref[...]
    y = jnp.zeros(out_ref.shape, jnp.float32)
    for q in range(4):
        uq = jnp.maximum(acc_ref[q] * dsa + b1[:, 16 * q:16 * (q + 1)], 0.0)
        y = y + jnp.dot(uq, w2[16 * q:16 * (q + 1)],
                        preferred_element_type=jnp.float32)
    out_ref[...] = y * dsb


def _l2_prep(acc, deg_relu, deg_scat, b1, w2pad):
    b1p = jnp.pad(b1[None, :], ((0, 7), (0, 0)))
    g = N // _BLK
    full = lambda a: pl.BlockSpec(a.shape, lambda i: (0,) * a.ndim)
    return pl.pallas_call(
        _l2_body,
        grid=(g,),
        in_specs=[
            pl.BlockSpec((4, _BLK, 16), lambda i: (0, i, 0)),
            pl.BlockSpec((_BLK, 1), lambda i: (i, 0)),
            pl.BlockSpec((_BLK, 1), lambda i: (i, 0)),
            full(b1p), full(w2pad),
        ],
        out_specs=pl.BlockSpec((_BLK, 16), lambda i: (i, 0)),
        out_shape=jax.ShapeDtypeStruct((N, 16), jnp.float32),
    )(acc, deg_relu, deg_scat, b1p, w2pad)


def _fin_body(acc2_ref, degu_ref, degi_ref, b2_ref, u_ref, i_ref):
    dsu = lax.rsqrt(jnp.maximum(degu_ref[...], 1.0))
    dsi = lax.rsqrt(jnp.maximum(degi_ref[...], 1.0))
    b2 = b2_ref[...]
    u_ref[...] = acc2_ref[0][:, :2] * dsu + b2[0:1, :2]
    i_ref[...] = acc2_ref[1][:, :2] * dsi + b2[1:2, :2]


def _finalize(acc2, deg_u, deg_i, b2u, b2i):
    b2 = jnp.pad(jnp.stack([b2u, b2i]), ((0, 6), (0, 0)))
    g = N // _BLK
    full = lambda a: pl.BlockSpec(a.shape, lambda i: (0,) * a.ndim)
    return pl.pallas_call(
        _fin_body,
        grid=(g,),
        in_specs=[
            pl.BlockSpec((2, _BLK, 16), lambda i: (0, i, 0)),
            pl.BlockSpec((_BLK, 1), lambda i: (i, 0)),
            pl.BlockSpec((_BLK, 1), lambda i: (i, 0)),
            full(b2),
        ],
        out_specs=[
            pl.BlockSpec((_BLK, 2), lambda i: (i, 0)),
            pl.BlockSpec((_BLK, 2), lambda i: (i, 0)),
        ],
        out_shape=[
            jax.ShapeDtypeStruct((N, 2), jnp.float32),
            jax.ShapeDtypeStruct((N, 2), jnp.float32),
        ],
    )(acc2, deg_u, deg_i, b2)


def kernel(x_user, x_item, params, edge_index_u2i, edge_index_i2u):
    p = params
    E = edge_index_u2i.shape[1]
    i32 = jnp.int32
    su_u = edge_index_u2i[0].astype(i32)
    di_u = edge_index_u2i[1].astype(i32)
    si_i = edge_index_i2u[0].astype(i32)
    du_i = edge_index_i2u[1].astype(i32)

    rows3 = lambda a: a.reshape(2, E // IB, IB)
    rows2 = lambda a: a.reshape(E // IB, IB)

    ones = jnp.ones((IR, IB), jnp.float32)
    zeros_deg = jnp.zeros((2000,), jnp.float32)
    degs = _sc_degrees(
        rows3(jnp.stack([su_u, si_i])),
        rows3(jnp.stack([di_u, du_i]) + N),
        ones, zeros_deg, E,
    ).reshape(2, 2 * N)
    deg_src_u2i = degs[0, :N].reshape(N, 1)
    deg_dst_u2i = degs[0, N:].reshape(N, 1)
    deg_src_i2u = degs[1, :N].reshape(N, 1)
    deg_dst_i2u = degs[1, N:].reshape(N, 1)

    t_i2u = _encode_project(x_item, p["enc_item"], deg_src_i2u, p["l1_i2u_W"])
    t_u2i = _encode_project(x_user, p["enc_user"], deg_src_u2i, p["l1_u2i_W"])

    zeros16 = jnp.zeros((1000, 16), jnp.float32)

    def scat64(table4, src, dst):
        tab = table4.reshape(4 * N, 16)
        dst_r = rows2(dst)
        parts = [
            _sc_scatter(tab, rows3(jnp.stack([src + 2 * q * N,
                                              src + (2 * q + 1) * N])),
                        dst_r, zeros16, 16, E, dst_per_core=False)[:, :N]
            for q in range(2)
        ]
        return jnp.concatenate(parts, axis=0)

    acc_u = scat64(t_i2u, si_i, du_i)
    acc_i = scat64(t_u2i, su_u, di_u)

    pad16 = lambda w: jnp.pad(w, ((0, 0), (0, 14)))
    y_i = _l2_prep(acc_i, deg_dst_u2i, deg_src_i2u, p["l1_u2i_b"],
                   pad16(p["l2_i2u_W"]))
    y_u = _l2_prep(acc_u, deg_dst_i2u, deg_src_u2i, p["l1_i2u_b"],
                   pad16(p["l2_u2i_W"]))

    acc2 = _sc_scatter(jnp.concatenate([y_i, y_u], axis=0),
                       rows3(jnp.stack([si_i, su_u + N])),
                       rows3(jnp.stack([du_i, di_u])),
                       zeros16, 16, E, dst_per_core=True)[:, :N]

    u2, i2 = _finalize(acc2, deg_dst_i2u, deg_dst_u2i,
                       p["l2_i2u_b"], p["l2_u2i_b"])
    return (u2, i2)


# speedup vs baseline: 22.3028x; 1.1360x over previous
import functools

import jax
import jax.numpy as jnp
from jax import lax
from jax.experimental import pallas as pl
from jax.experimental.pallas import tpu as pltpu
from jax.experimental.pallas import tpu_sc as plsc

N = 50000
D = 64
NTILE = 16
CHUNK = 1000
GRP = 10
IB = 1000
IR = 10

def _mesh():
    return plsc.VectorSubcoreMesh(core_axis_name="c", subcore_axis_name="s")


_SC_PARAMS = pltpu.CompilerParams(use_tc_tiling_on_sc=False)


def _sc_degrees(srcs, dsts, ones, zeros, E):
    rows_per_tile = E // (NTILE * IB)
    n_chunks = rows_per_tile // IR

    @functools.partial(
        pl.kernel,
        mesh=_mesh(),
        compiler_params=_SC_PARAMS,
        out_type=jax.ShapeDtypeStruct((4 * N,), jnp.float32),
        scratch_types=[
            pltpu.VMEM((IR, IB), jnp.int32),
            pltpu.VMEM((IR, IB), jnp.int32),
            pltpu.VMEM((IR, IB), jnp.float32),
            pltpu.VMEM((2000,), jnp.float32),
            pltpu.VMEM_SHARED((2 * N,), jnp.float32),
        ],
    )
    def k(srcs_hbm, dsts_hbm, ones_hbm, zeros_hbm, out_hbm, sidx, didx, ones_v,
          zv, acc):
        c = lax.axis_index("c")
        s = lax.axis_index("s")
        pltpu.sync_copy(ones_hbm, ones_v)
        pltpu.sync_copy(zeros_hbm, zv)

        @pl.when(s < 4)
        def _():
            base = s * (2 * N // 4)
            for kk in range(12):
                pltpu.sync_copy(zv, acc.at[pl.ds(base + kk * 2000, 2000)])
            pltpu.sync_copy(zv.at[pl.ds(0, 1000)],
                            acc.at[pl.ds(base + 24000, 1000)])

        plsc.subcore_barrier()

        def chunk(i, carry):
            r0 = s * rows_per_tile + i * IR
            pltpu.sync_copy(srcs_hbm.at[c, pl.ds(r0, IR)], sidx)
            pltpu.sync_copy(dsts_hbm.at[c, pl.ds(r0, IR)], didx)
            for j in range(IR):
                pltpu.sync_copy(ones_v.at[j], acc.at[sidx.at[j]], add=True)
                pltpu.sync_copy(ones_v.at[j], acc.at[didx.at[j]], add=True)
            return carry

        lax.fori_loop(0, n_chunks, chunk, 0)
        plsc.subcore_barrier()

        @pl.when(s < 4)
        def _():
            base = s * (2 * N // 4)
            for kk in range(12):
                pltpu.sync_copy(acc.at[pl.ds(base + kk * 2000, 2000)], zv)
                pltpu.sync_copy(
                    zv, out_hbm.at[pl.ds(c * 2 * N + base + kk * 2000, 2000)])
            pltpu.sync_copy(acc.at[pl.ds(base + 24000, 1000)],
                            zv.at[pl.ds(0, 1000)])
            pltpu.sync_copy(zv.at[pl.ds(0, 1000)],
                            out_hbm.at[pl.ds(c * 2 * N + base + 24000, 1000)])

    return k(srcs, dsts, ones, zeros)


NP = 50048


def _sc_scatter(table, srcs, dsts, zeros, W, E, dst_per_core):
    n_chunks = E // (NTILE * CHUNK)
    rpt = NP // NTILE

    @functools.partial(
        pl.kernel,
        mesh=_mesh(),
        compiler_params=_SC_PARAMS,
        out_type=jax.ShapeDtypeStruct((2, NP, W), jnp.float32),
        scratch_types=[
            pltpu.VMEM((GRP, CHUNK), jnp.int32),
            pltpu.VMEM((GRP, CHUNK), jnp.int32),
            pltpu.VMEM((2, CHUNK, W), jnp.float32),
            pltpu.VMEM((1000, W), jnp.float32),
            pltpu.VMEM_SHARED((NP, W), jnp.float32),
            pltpu.SemaphoreType.DMA,
            pltpu.SemaphoreType.DMA,
        ],
    )
    def k(table_hbm, srcs_hbm, dsts_hbm, zeros_hbm, out_hbm, sidx, didx, rows,
          zbuf, acc, semA, semB):
        c = lax.axis_index("c")
        s = lax.axis_index("s")
        base = s * rpt
        pltpu.sync_copy(zeros_hbm, zbuf)
        for kk in range(3):
            pltpu.sync_copy(zbuf, acc.at[pl.ds(base + kk * 1000, 1000)])
        pltpu.sync_copy(zbuf.at[pl.ds(0, 128)], acc.at[pl.ds(base + 3000, 128)])
        plsc.subcore_barrier()

        sems = (semA, semB)
        g0 = s * n_chunks
        n_groups = n_chunks // GRP

        def group(g, carry):
            pltpu.sync_copy(srcs_hbm.at[c, pl.ds(g0 + g * GRP, GRP)], sidx)
            if dst_per_core:
                pltpu.sync_copy(dsts_hbm.at[c, pl.ds(g0 + g * GRP, GRP)], didx)
            else:
                pltpu.sync_copy(dsts_hbm.at[pl.ds(g0 + g * GRP, GRP)], didx)
            pltpu.async_copy(table_hbm.at[sidx.at[0]], rows.at[0], sems[0])
            for j in range(GRP):
                b = j % 2
                if j + 1 < GRP:
                    pltpu.async_copy(table_hbm.at[sidx.at[j + 1]],
                                     rows.at[1 - b], sems[1 - b])
                pltpu.make_async_copy(table_hbm.at[sidx.at[j]],
                                      rows.at[b], sems[b]).wait()
                pltpu.sync_copy(rows.at[b], acc.at[didx.at[j]], add=True)
            return carry

        lax.fori_loop(0, n_groups, group, 0)
        plsc.subcore_barrier()
        for kk in range(3):
            pltpu.sync_copy(acc.at[pl.ds(base + kk * 1000, 1000)], zbuf)
            pltpu.sync_copy(zbuf, out_hbm.at[c, pl.ds(base + kk * 1000, 1000)])
        pltpu.sync_copy(acc.at[pl.ds(base + 3000, 128)], zbuf.at[pl.ds(0, 128)])
        pltpu.sync_copy(zbuf.at[pl.ds(0, 128)],
                        out_hbm.at[c, pl.ds(base + 3000, 128)])

    return k(table, srcs, dsts, zeros)


_BLK = 2000


def _ln_in(x, g, b):
    mu = jnp.mean(x, axis=-1, keepdims=True)
    var = jnp.mean((x - mu) ** 2, axis=-1, keepdims=True)
    return (x - mu) * lax.rsqrt(var + 1e-5) * g + b


def _enc_body(x_ref, win_ref, w1s_ref, w2s_ref, vecs_ref, wout_ref, wg_ref,
              deg_ref, out_ref):
    f32 = jnp.float32
    x = x_ref[...]
    v = vecs_ref
    h = jnp.dot(x, win_ref[...], preferred_element_type=f32) + v[0]
    for i in range(4):
        z = _ln_in(h, v[1 + i], v[5 + i])
        z = jnp.maximum(jnp.dot(z, w1s_ref[i], preferred_element_type=f32) + v[9 + i], 0.0)
        h = h + jnp.dot(z, w2s_ref[i], preferred_element_type=f32) + v[13 + i]
    h = jnp.maximum(_ln_in(h, v[17], v[18]), 0.0)
    h = jnp.dot(h, wout_ref[...], preferred_element_type=f32) + v[19]
    dsc = lax.rsqrt(jnp.maximum(deg_ref[...], 1.0))
    hw = jnp.dot(h * dsc, wg_ref[...], preferred_element_type=f32)
    for q in range(4):
        out_ref[q] = hw[:, 16 * q:16 * (q + 1)]


def _encode_project(x, enc, deg, wg):
    vecs = jnp.stack(
        [enc["b_in"]]
        + [b["g"] for b in enc["blocks"]]
        + [b["bt"] for b in enc["blocks"]]
        + [b["b1"] for b in enc["blocks"]]
        + [b["b2"] for b in enc["blocks"]]
        + [enc["g_out"], enc["bt_out"], enc["b_out"]]
    )
    vecs = jnp.pad(vecs, ((0, 4), (0, 0)))
    w1s = jnp.stack([b["W1"] for b in enc["blocks"]])
    w2s = jnp.stack([b["W2"] for b in enc["blocks"]])
    g = N // _BLK
    full = lambda a: pl.BlockSpec(a.shape, lambda i: (0,) * a.ndim)
    return pl.pallas_call(
        _enc_body,
        grid=(g,),
        in_specs=[
            pl.BlockSpec((_BLK, x.shape[1]), lambda i: (i, 0)),
            full(enc["W_in"]), full(w1s), full(w2s), full(vecs),
            full(enc["W_out"]), full(wg),
            pl.BlockSpec((_BLK, 1), lambda i: (i, 0)),
        ],
        out_specs=pl.BlockSpec((4, _BLK, 16), lambda i: (0, i, 0)),
        out_shape=jax.ShapeDtypeStruct((4, N, 16), jnp.float32),
    )(x, enc["W_in"], w1s, w2s, vecs, enc["W_out"], wg, deg)


def _l2_body(acc_ref, dega_ref, degb_ref, b1_ref, w2_ref, out_ref):
    dsa = lax.rsqrt(jnp.maximum(dega_ref[...], 1.0))
    dsb = lax.rsqrt(jnp.maximum(degb_ref[...], 1.0))
    b1 = b1_ref[0:1]
    w2 = w2_ref[...]
    y = jnp.zeros(out_ref.shape, jnp.float32)
    for q in range(4):
        uq = jnp.maximum(acc_ref[q] * dsa + b1[:, 16 * q:16 * (q + 1)], 0.0)
        y = y + jnp.dot(uq, w2[16 * q:16 * (q + 1)],
                        preferred_element_type=jnp.float32)
    out_ref[...] = y * dsb


def _l2_prep(acc, deg_relu, deg_scat, b1, w2pad):
    b1p = jnp.pad(b1[None, :], ((0, 7), (0, 0)))
    g = N // _BLK
    full = lambda a: pl.BlockSpec(a.shape, lambda i: (0,) * a.ndim)
    return pl.pallas_call(
        _l2_body,
        grid=(g,),
        in_specs=[
            pl.BlockSpec((4, _BLK, 16), lambda i: (0, i, 0)),
            pl.BlockSpec((_BLK, 1), lambda i: (i, 0)),
            pl.BlockSpec((_BLK, 1), lambda i: (i, 0)),
            full(b1p), full(w2pad),
        ],
        out_specs=pl.BlockSpec((_BLK, 16), lambda i: (i, 0)),
        out_shape=jax.ShapeDtypeStruct((N, 16), jnp.float32),
    )(acc, deg_relu, deg_scat, b1p, w2pad)


def _fin_body(acc2_ref, degu_ref, degi_ref, b2_ref, u_ref, i_ref):
    dsu = lax.rsqrt(jnp.maximum(degu_ref[...], 1.0))
    dsi = lax.rsqrt(jnp.maximum(degi_ref[...], 1.0))
    b2 = b2_ref[...]
    u_ref[...] = acc2_ref[0][:, :2] * dsu + b2[0:1, :2]
    i_ref[...] = acc2_ref[1][:, :2] * dsi + b2[1:2, :2]


def _finalize(acc2, deg_u, deg_i, b2u, b2i):
    b2 = jnp.pad(jnp.stack([b2u, b2i]), ((0, 6), (0, 0)))
    g = N // _BLK
    full = lambda a: pl.BlockSpec(a.shape, lambda i: (0,) * a.ndim)
    return pl.pallas_call(
        _fin_body,
        grid=(g,),
        in_specs=[
            pl.BlockSpec((2, _BLK, 16), lambda i: (0, i, 0)),
            pl.BlockSpec((_BLK, 1), lambda i: (i, 0)),
            pl.BlockSpec((_BLK, 1), lambda i: (i, 0)),
            full(b2),
        ],
        out_specs=[
            pl.BlockSpec((_BLK, 2), lambda i: (i, 0)),
            pl.BlockSpec((_BLK, 2), lambda i: (i, 0)),
        ],
        out_shape=[
            jax.ShapeDtypeStruct((N, 2), jnp.float32),
            jax.ShapeDtypeStruct((N, 2), jnp.float32),
        ],
    )(acc2, deg_u, deg_i, b2)


def kernel(x_user, x_item, params, edge_index_u2i, edge_index_i2u):
    p = params
    E = edge_index_u2i.shape[1]
    i32 = jnp.int32
    su_u = edge_index_u2i[0].astype(i32)
    di_u = edge_index_u2i[1].astype(i32)
    si_i = edge_index_i2u[0].astype(i32)
    du_i = edge_index_i2u[1].astype(i32)

    rows3 = lambda a: a.reshape(2, E // IB, IB)
    rows3c = lambda a: a.reshape(2, E // CHUNK, CHUNK)
    rows2c = lambda a: a.reshape(E // CHUNK, CHUNK)

    ones = jnp.ones((IR, IB), jnp.float32)
    zeros_deg = jnp.zeros((2000,), jnp.float32)
    degs = _sc_degrees(
        rows3(jnp.stack([su_u, si_i])),
        rows3(jnp.stack([di_u, du_i]) + N),
        ones, zeros_deg, E,
    ).reshape(2, 2 * N)
    deg_src_u2i = degs[0, :N].reshape(N, 1)
    deg_dst_u2i = degs[0, N:].reshape(N, 1)
    deg_src_i2u = degs[1, :N].reshape(N, 1)
    deg_dst_i2u = degs[1, N:].reshape(N, 1)

    t_i2u = _encode_project(x_item, p["enc_item"], deg_src_i2u, p["l1_i2u_W"])
    t_u2i = _encode_project(x_user, p["enc_user"], deg_src_u2i, p["l1_u2i_W"])

    zeros16 = jnp.zeros((1000, 16), jnp.float32)

    def scat64(table4, src, dst):
        tab = table4.reshape(4 * N, 16)
        dst_r = rows2c(dst)
        parts = [
            _sc_scatter(tab, rows3c(jnp.stack([src + 2 * q * N,
                                               src + (2 * q + 1) * N])),
                        dst_r, zeros16, 16, E, dst_per_core=False)[:, :N]
            for q in range(2)
        ]
        return jnp.concatenate(parts, axis=0)

    acc_u = scat64(t_i2u, si_i, du_i)
    acc_i = scat64(t_u2i, su_u, di_u)

    pad16 = lambda w: jnp.pad(w, ((0, 0), (0, 14)))
    y_i = _l2_prep(acc_i, deg_dst_u2i, deg_src_i2u, p["l1_u2i_b"],
                   pad16(p["l2_i2u_W"]))
    y_u = _l2_prep(acc_u, deg_dst_i2u, deg_src_u2i, p["l1_i2u_b"],
                   pad16(p["l2_u2i_W"]))

    acc2 = _sc_scatter(jnp.concatenate([y_i, y_u], axis=0),
                       rows3c(jnp.stack([si_i, su_u + N])),
                       rows3c(jnp.stack([du_i, di_u])),
                       zeros16, 16, E, dst_per_core=True)[:, :N]

    u2, i2 = _finalize(acc2, deg_dst_i2u, deg_dst_u2i,
                       p["l2_i2u_b"], p["l2_u2i_b"])
    return (u2, i2)
